# speedup vs baseline: 1.9615x; 1.9615x over previous
import functools

import jax
import jax.numpy as jnp
from jax.experimental import pallas as pl
from jax.experimental.pallas import tpu as pltpu


def _pool_kernel(x_ref, a_ref, o_ref):
    a = a_ref[...]
    deg = jnp.sum(a, axis=1, keepdims=True)
    deg = deg + (deg == 0).astype(jnp.float32)
    acc = jax.lax.dot_general(
        a,
        x_ref[...],
        (((1,), (0,)), ((), ())),
        preferred_element_type=jnp.float32,
    )
    o_ref[...] = acc / deg


@functools.partial(jax.jit, static_argnames=("block_m",))
def _mean_pool(inputs, adj, block_m=400):
    n, d = inputs.shape
    grid = (n // block_m,)
    return pl.pallas_call(
        _pool_kernel,
        grid=grid,
        in_specs=[
            pl.BlockSpec((n, d), lambda i: (0, 0)),
            pl.BlockSpec((block_m, n), lambda i: (i, 0)),
        ],
        out_specs=pl.BlockSpec((block_m, d), lambda i: (i, 0)),
        out_shape=jax.ShapeDtypeStruct((n, d), jnp.float32),
        compiler_params=pltpu.CompilerParams(
            dimension_semantics=("arbitrary",),
        ),
    )(inputs, adj)


def kernel(inputs, adj):
    return _mean_pool(inputs, adj)
